# baseline (speedup 1.0000x reference)
import dataclasses

import jax
import jax.numpy as jnp
from jax import lax
from jax.experimental import pallas as pl
from jax.experimental.pallas import tpu as pltpu
from jax.experimental.pallas import tpu_sc as plsc

NEG = 0.2
H = 4
HC = 128
NC, NS = 2, 16
NW = NC * NS
EB = 48
NBLK = 214
EPT = EB * NBLK
EPAD = NW * EPT
NPAD = 10240
DROWS = NPAD // 8


def _mm_body(f_ref, wl_ref, wr_ref, xl_ref, xr_ref):
    f = f_ref[...]
    xl_ref[...] = jnp.dot(f, wl_ref[...], preferred_element_type=jnp.float32)
    xr_ref[...] = jnp.dot(f, wr_ref[...], preferred_element_type=jnp.float32)


def _project(feats, W_l, W_r):
    n, d = feats.shape
    rb = 1280
    io_spec = pl.BlockSpec((rb, HC), lambda i: (i, 0))
    w_spec = pl.BlockSpec((d, HC), lambda i: (0, 0))
    return pl.pallas_call(
        _mm_body,
        grid=(n // rb,),
        in_specs=[pl.BlockSpec((rb, d), lambda i: (i, 0)), w_spec, w_spec],
        out_specs=[io_spec, io_spec],
        out_shape=[
            jax.ShapeDtypeStruct((n, HC), jnp.float32),
            jax.ShapeDtypeStruct((n, HC), jnp.float32),
        ],
    )(feats, W_l, W_r)


def _bcast_lane(v, j):
    idx = jnp.full((16, 1), j, jnp.int32)
    dn = lax.GatherDimensionNumbers(
        offset_dims=(), collapsed_slice_dims=(0,), start_index_map=(0,))
    return lax.gather(v, idx, dn, (1,),
                      mode=lax.GatherScatterMode.PROMISE_IN_BOUNDS)


def _edge_pass(xl, xr, src, dst, att_flat):
    rpt = NPAD // NS

    mesh = plsc.VectorSubcoreMesh(core_axis_name="c", subcore_axis_name="s")

    def body(xl_hbm, xr_hbm, src_hbm, dst_hbm, att_hbm, msg_hbm, den_hbm,
             si0, si1, di0, di1, xl0, xl1, xr0, xr1, mb, mb2, didx2, attv,
             acc, accd, sg0, sg1):
        c = lax.axis_index("c")
        s = lax.axis_index("s")
        wid = c * NS + s
        ebase = wid * EPT

        pltpu.sync_copy(att_hbm, attv)
        att_regs = [attv[pl.ds(16 * k, 16)] for k in range(8)]
        lane = lax.iota(jnp.int32, 16)
        masks = [(lane == h).astype(jnp.float32) for h in range(H)]

        zeros16 = jnp.zeros((16,), jnp.float32)

        @pl.loop(0, EB)
        def _(r):
            for k in range(HC // 16):
                mb[r, pl.ds(16 * k, 16)] = zeros16

        rbase = s * rpt
        nz = rpt // EB
        for t in range(nz):
            pltpu.sync_copy(mb.at[pl.ds(0, EB)],
                            acc.at[pl.ds(rbase + t * EB, EB)])
        rem = rpt - nz * EB
        if rem:
            pltpu.sync_copy(mb.at[pl.ds(0, rem)],
                            acc.at[pl.ds(rbase + nz * EB, rem)])
        dbase = s * (DROWS // NS)
        pltpu.sync_copy(mb.at[pl.ds(0, EB)], accd.at[pl.ds(dbase, EB)])
        pltpu.sync_copy(mb.at[pl.ds(0, DROWS // NS - EB)],
                        accd.at[pl.ds(dbase + EB, DROWS // NS - EB)])
        plsc.subcore_barrier()

        def fetch(j, sib, dib, xlb, xrb, sem):
            pltpu.sync_copy(src_hbm.at[pl.ds(ebase + j * EB, EB)], sib)
            pltpu.sync_copy(dst_hbm.at[pl.ds(ebase + j * EB, EB)], dib)
            pltpu.async_copy(xl_hbm.at[sib], xlb, sem)
            pltpu.async_copy(xr_hbm.at[dib], xrb, sem)

        def wait(sib, dib, xlb, xrb, sem):
            pltpu.make_async_copy(xl_hbm.at[sib], xlb, sem).wait()
            pltpu.make_async_copy(xr_hbm.at[dib], xrb, sem).wait()

        def process(xlb, xrb, dib):
            @plsc.parallel_loop(0, EB, step=16, unroll=2)
            def _(g):
                dvec = dib[pl.ds(g, 16)]
                didx2[pl.ds(g, 16)] = lax.shift_right_logical(dvec, 3)
                for j in range(16):
                    r = g + j
                    den_vec = zeros16
                    for h in range(H):
                        a0 = xlb[r, pl.ds(32 * h, 16)]
                        a1 = xlb[r, pl.ds(32 * h + 16, 16)]
                        b0 = xrb[r, pl.ds(32 * h, 16)]
                        b1 = xrb[r, pl.ds(32 * h + 16, 16)]
                        z0 = a0 + b0
                        z1 = a1 + b1
                        l0 = jnp.maximum(z0, NEG * z0)
                        l1 = jnp.maximum(z1, NEG * z1)
                        t = l0 * att_regs[2 * h] + l1 * att_regs[2 * h + 1]
                        sc = jnp.sum(t)
                        w = jnp.exp(jnp.full((16,), sc, jnp.float32))
                        mb[r, pl.ds(32 * h, 16)] = a0 * w
                        mb[r, pl.ds(32 * h + 16, 16)] = a1 * w
                        den_vec = den_vec + masks[h] * w
                    grp = jnp.bitwise_and(_bcast_lane(dvec, j), 7)
                    for k in range(8):
                        vk = jnp.where(grp == k, den_vec, zeros16)
                        mb2[r, pl.ds(16 * k, 16)] = vk

            pltpu.sync_copy(mb, acc.at[dib], add=True)
            pltpu.sync_copy(mb2, accd.at[didx2], add=True)

        bufs = ((si0, di0, xl0, xr0, sg0),
                (si1, di1, xl1, xr1, sg1))

        fetch(0, *bufs[0])

        @pl.loop(0, NBLK, step=2)
        def _(jj):
            for b in (0, 1):
                sib, dib, xlb, xrb, sem = bufs[b]
                if b == 0:
                    fetch(jj + 1, *bufs[1])
                else:
                    @pl.when(jj + 2 < NBLK)
                    def _():
                        fetch(jj + 2, *bufs[0])
                wait(sib, dib, xlb, xrb, sem)
                process(xlb, xrb, dib)

        plsc.subcore_barrier()
        pltpu.sync_copy(acc.at[pl.ds(rbase, rpt)],
                        msg_hbm.at[c, pl.ds(rbase, rpt)])
        drpt = DROWS // NS
        pltpu.sync_copy(accd.at[pl.ds(dbase, drpt)],
                        den_hbm.at[c, pl.ds(dbase, drpt)])

    f32 = jnp.float32
    i32 = jnp.int32
    cp = pltpu.CompilerParams()
    if "needs_layout_passes" in pltpu.CompilerParams.__dataclass_fields__:
        cp = dataclasses.replace(cp, needs_layout_passes=False)
    return pl.kernel(
        body,
        compiler_params=cp,
        out_type=(
            jax.ShapeDtypeStruct((NC, NPAD, HC), f32),
            jax.ShapeDtypeStruct((NC, DROWS, HC), f32),
        ),
        mesh=mesh,
        scratch_types=[
            pltpu.VMEM((EB,), i32), pltpu.VMEM((EB,), i32),
            pltpu.VMEM((EB,), i32), pltpu.VMEM((EB,), i32),
            pltpu.VMEM((EB, HC), f32), pltpu.VMEM((EB, HC), f32),
            pltpu.VMEM((EB, HC), f32), pltpu.VMEM((EB, HC), f32),
            pltpu.VMEM((EB, HC), f32), pltpu.VMEM((EB, HC), f32),
            pltpu.VMEM((EB,), i32),
            pltpu.VMEM((HC,), f32),
            pltpu.VMEM_SHARED((NPAD, HC), f32),
            pltpu.VMEM_SHARED((DROWS, HC), f32),
            pltpu.SemaphoreType.DMA, pltpu.SemaphoreType.DMA,
        ],
    )(xl, xr, src, dst, att_flat)


def _post_body(xl_ref, xr_ref, a0_ref, a1_ref, d0_ref, d1_ref, att_ref,
               bias_ref, o_ref):
    xl = xl_ref[...]
    xr = xr_ref[...]
    z = xl + xr
    lk = jnp.maximum(z, NEG * z)
    t = lk * att_ref[...]
    a0 = a0_ref[...]
    a1 = a1_ref[...]
    for h in range(H):
        sl = slice(32 * h, 32 * h + 32)
        s_h = jnp.sum(t[:, sl], axis=1, keepdims=True)
        w_h = jnp.exp(s_h)
        num = a0[:, sl] + a1[:, sl] + w_h * xl[:, sl]
        den = d0_ref[:, h:h + 1] + d1_ref[:, h:h + 1] + w_h
        o_ref[:, sl] = num / den + bias_ref[:, sl]


def _epilogue(xl, xr, a0, a1, d0, d1, att_row, bias_row):
    n = xl.shape[0]
    rb = 1000
    return pl.pallas_call(
        _post_body,
        grid=(n // rb,),
        in_specs=[
            pl.BlockSpec((rb, HC), lambda i: (i, 0)),
            pl.BlockSpec((rb, HC), lambda i: (i, 0)),
            pl.BlockSpec((rb, HC), lambda i: (i, 0)),
            pl.BlockSpec((rb, HC), lambda i: (i, 0)),
            pl.BlockSpec((rb, 16), lambda i: (i, 0)),
            pl.BlockSpec((rb, 16), lambda i: (i, 0)),
            pl.BlockSpec((1, HC), lambda i: (0, 0)),
            pl.BlockSpec((1, HC), lambda i: (0, 0)),
        ],
        out_specs=pl.BlockSpec((rb, HC), lambda i: (i, 0)),
        out_shape=jax.ShapeDtypeStruct((n, HC), jnp.float32),
    )(xl, xr, a0, a1, d0, d1, att_row, bias_row)


def kernel(feats, edges, batches, W_l, W_r, att, bias):
    n = feats.shape[0]
    e = edges.shape[1]
    feats_p = jnp.pad(feats, ((0, NPAD - n), (0, 0)))
    pad_idx = jnp.full((EPAD - e,), NPAD - 1, jnp.int32)
    srcp = jnp.concatenate([edges[0], pad_idx])
    dstp = jnp.concatenate([edges[1], pad_idx])
    xl, xr = _project(feats_p, W_l, W_r)
    acc, accd = _edge_pass(xl, xr, srcp, dstp, att.reshape(-1))
    den = accd.reshape(NC, NPAD, 16)[:, :n, :]
    out = _epilogue(xl[:n], xr[:n], acc[0, :n], acc[1, :n], den[0], den[1],
                    att.reshape(1, -1), bias.reshape(1, -1))
    return out


# speedup vs baseline: 1.0028x; 1.0028x over previous
import dataclasses

import jax
import jax.numpy as jnp
from jax import lax
from jax.experimental import pallas as pl
from jax.experimental.pallas import tpu as pltpu
from jax.experimental.pallas import tpu_sc as plsc

NEG = 0.2
H = 4
HC = 128
NC, NS = 2, 16
NW = NC * NS
EB = 48
NBLK = 214
EPT = EB * NBLK
EPAD = NW * EPT
NPAD = 10240
DROWS = NPAD // 8


def _mm_body(f_ref, wl_ref, wr_ref, xl_ref, xr_ref):
    f = f_ref[...]
    xl_ref[...] = jnp.dot(f, wl_ref[...], preferred_element_type=jnp.float32)
    xr_ref[...] = jnp.dot(f, wr_ref[...], preferred_element_type=jnp.float32)


def _project(feats, W_l, W_r):
    n, d = feats.shape
    rb = 1280
    io_spec = pl.BlockSpec((rb, HC), lambda i: (i, 0))
    w_spec = pl.BlockSpec((d, HC), lambda i: (0, 0))
    return pl.pallas_call(
        _mm_body,
        grid=(n // rb,),
        in_specs=[pl.BlockSpec((rb, d), lambda i: (i, 0)), w_spec, w_spec],
        out_specs=[io_spec, io_spec],
        out_shape=[
            jax.ShapeDtypeStruct((n, HC), jnp.float32),
            jax.ShapeDtypeStruct((n, HC), jnp.float32),
        ],
    )(feats, W_l, W_r)


def _bcast_lane(v, j):
    idx = jnp.full((16, 1), j, jnp.int32)
    dn = lax.GatherDimensionNumbers(
        offset_dims=(), collapsed_slice_dims=(0,), start_index_map=(0,))
    return lax.gather(v, idx, dn, (1,),
                      mode=lax.GatherScatterMode.PROMISE_IN_BOUNDS)


def _edge_pass(xl, xr, src, dst, att_flat):
    rpt = NPAD // NS

    mesh = plsc.VectorSubcoreMesh(core_axis_name="c", subcore_axis_name="s")

    def body(xl_hbm, xr_hbm, src_hbm, dst_hbm, att_hbm, msg_hbm, den_hbm,
             si0, si1, di0, di1, xl0, xl1, xr0, xr1, mb, mb2, didx2, attv,
             acc, accd, sg0, sg1):
        c = lax.axis_index("c")
        s = lax.axis_index("s")
        wid = c * NS + s
        ebase = wid * EPT

        pltpu.sync_copy(att_hbm, attv)
        att_regs = [attv[pl.ds(16 * k, 16)] for k in range(8)]
        lane = lax.iota(jnp.int32, 16)
        masks = [(lane == h).astype(jnp.float32) for h in range(H)]

        zeros16 = jnp.zeros((16,), jnp.float32)

        @pl.loop(0, EB)
        def _(r):
            for k in range(HC // 16):
                mb[r, pl.ds(16 * k, 16)] = zeros16

        rbase = s * rpt
        nz = rpt // EB
        for t in range(nz):
            pltpu.sync_copy(mb.at[pl.ds(0, EB)],
                            acc.at[pl.ds(rbase + t * EB, EB)])
        rem = rpt - nz * EB
        if rem:
            pltpu.sync_copy(mb.at[pl.ds(0, rem)],
                            acc.at[pl.ds(rbase + nz * EB, rem)])
        dbase = s * (DROWS // NS)
        pltpu.sync_copy(mb.at[pl.ds(0, EB)], accd.at[pl.ds(dbase, EB)])
        pltpu.sync_copy(mb.at[pl.ds(0, DROWS // NS - EB)],
                        accd.at[pl.ds(dbase + EB, DROWS // NS - EB)])
        plsc.subcore_barrier()

        def fetch(j, sib, dib, xlb, xrb, sem):
            pltpu.sync_copy(src_hbm.at[pl.ds(ebase + j * EB, EB)], sib)
            pltpu.sync_copy(dst_hbm.at[pl.ds(ebase + j * EB, EB)], dib)
            pltpu.async_copy(xl_hbm.at[sib], xlb, sem)
            pltpu.async_copy(xr_hbm.at[dib], xrb, sem)

        def wait(sib, dib, xlb, xrb, sem):
            pltpu.make_async_copy(xl_hbm.at[sib], xlb, sem).wait()
            pltpu.make_async_copy(xr_hbm.at[dib], xrb, sem).wait()

        def process(xlb, xrb, dib):
            for g in range(0, EB, 16):
                dvec = dib[pl.ds(g, 16)]
                didx2[pl.ds(g, 16)] = lax.shift_right_logical(dvec, 3)
                for j in range(16):
                    r = g + j
                    den_vec = zeros16
                    for h in range(H):
                        a0 = xlb[r, pl.ds(32 * h, 16)]
                        a1 = xlb[r, pl.ds(32 * h + 16, 16)]
                        b0 = xrb[r, pl.ds(32 * h, 16)]
                        b1 = xrb[r, pl.ds(32 * h + 16, 16)]
                        z0 = a0 + b0
                        z1 = a1 + b1
                        l0 = jnp.maximum(z0, NEG * z0)
                        l1 = jnp.maximum(z1, NEG * z1)
                        t = l0 * att_regs[2 * h] + l1 * att_regs[2 * h + 1]
                        sc = jnp.sum(t)
                        w = jnp.exp(jnp.full((16,), sc, jnp.float32))
                        mb[r, pl.ds(32 * h, 16)] = a0 * w
                        mb[r, pl.ds(32 * h + 16, 16)] = a1 * w
                        den_vec = den_vec + masks[h] * w
                    grp = jnp.bitwise_and(_bcast_lane(dvec, j), 7)
                    for k in range(8):
                        vk = jnp.where(grp == k, den_vec, zeros16)
                        mb2[r, pl.ds(16 * k, 16)] = vk

            pltpu.sync_copy(mb, acc.at[dib], add=True)
            pltpu.sync_copy(mb2, accd.at[didx2], add=True)

        bufs = ((si0, di0, xl0, xr0, sg0),
                (si1, di1, xl1, xr1, sg1))

        fetch(0, *bufs[0])

        @pl.loop(0, NBLK, step=2)
        def _(jj):
            for b in (0, 1):
                sib, dib, xlb, xrb, sem = bufs[b]
                if b == 0:
                    fetch(jj + 1, *bufs[1])
                else:
                    @pl.when(jj + 2 < NBLK)
                    def _():
                        fetch(jj + 2, *bufs[0])
                wait(sib, dib, xlb, xrb, sem)
                process(xlb, xrb, dib)

        plsc.subcore_barrier()
        pltpu.sync_copy(acc.at[pl.ds(rbase, rpt)],
                        msg_hbm.at[c, pl.ds(rbase, rpt)])
        drpt = DROWS // NS
        pltpu.sync_copy(accd.at[pl.ds(dbase, drpt)],
                        den_hbm.at[c, pl.ds(dbase, drpt)])

    f32 = jnp.float32
    i32 = jnp.int32
    cp = pltpu.CompilerParams()
    if "needs_layout_passes" in pltpu.CompilerParams.__dataclass_fields__:
        cp = dataclasses.replace(cp, needs_layout_passes=False)
    return pl.kernel(
        body,
        compiler_params=cp,
        out_type=(
            jax.ShapeDtypeStruct((NC, NPAD, HC), f32),
            jax.ShapeDtypeStruct((NC, DROWS, HC), f32),
        ),
        mesh=mesh,
        scratch_types=[
            pltpu.VMEM((EB,), i32), pltpu.VMEM((EB,), i32),
            pltpu.VMEM((EB,), i32), pltpu.VMEM((EB,), i32),
            pltpu.VMEM((EB, HC), f32), pltpu.VMEM((EB, HC), f32),
            pltpu.VMEM((EB, HC), f32), pltpu.VMEM((EB, HC), f32),
            pltpu.VMEM((EB, HC), f32), pltpu.VMEM((EB, HC), f32),
            pltpu.VMEM((EB,), i32),
            pltpu.VMEM((HC,), f32),
            pltpu.VMEM_SHARED((NPAD, HC), f32),
            pltpu.VMEM_SHARED((DROWS, HC), f32),
            pltpu.SemaphoreType.DMA, pltpu.SemaphoreType.DMA,
        ],
    )(xl, xr, src, dst, att_flat)


def _post_body(xl_ref, xr_ref, a0_ref, a1_ref, d0_ref, d1_ref, att_ref,
               bias_ref, o_ref):
    xl = xl_ref[...]
    xr = xr_ref[...]
    z = xl + xr
    lk = jnp.maximum(z, NEG * z)
    t = lk * att_ref[...]
    a0 = a0_ref[...]
    a1 = a1_ref[...]
    for h in range(H):
        sl = slice(32 * h, 32 * h + 32)
        s_h = jnp.sum(t[:, sl], axis=1, keepdims=True)
        w_h = jnp.exp(s_h)
        num = a0[:, sl] + a1[:, sl] + w_h * xl[:, sl]
        den = d0_ref[:, h:h + 1] + d1_ref[:, h:h + 1] + w_h
        o_ref[:, sl] = num / den + bias_ref[:, sl]


def _epilogue(xl, xr, a0, a1, d0, d1, att_row, bias_row):
    n = xl.shape[0]
    rb = 1000
    return pl.pallas_call(
        _post_body,
        grid=(n // rb,),
        in_specs=[
            pl.BlockSpec((rb, HC), lambda i: (i, 0)),
            pl.BlockSpec((rb, HC), lambda i: (i, 0)),
            pl.BlockSpec((rb, HC), lambda i: (i, 0)),
            pl.BlockSpec((rb, HC), lambda i: (i, 0)),
            pl.BlockSpec((rb, 16), lambda i: (i, 0)),
            pl.BlockSpec((rb, 16), lambda i: (i, 0)),
            pl.BlockSpec((1, HC), lambda i: (0, 0)),
            pl.BlockSpec((1, HC), lambda i: (0, 0)),
        ],
        out_specs=pl.BlockSpec((rb, HC), lambda i: (i, 0)),
        out_shape=jax.ShapeDtypeStruct((n, HC), jnp.float32),
    )(xl, xr, a0, a1, d0, d1, att_row, bias_row)


def kernel(feats, edges, batches, W_l, W_r, att, bias):
    n = feats.shape[0]
    e = edges.shape[1]
    feats_p = jnp.pad(feats, ((0, NPAD - n), (0, 0)))
    pad_idx = jnp.full((EPAD - e,), NPAD - 1, jnp.int32)
    srcp = jnp.concatenate([edges[0], pad_idx])
    dstp = jnp.concatenate([edges[1], pad_idx])
    xl, xr = _project(feats_p, W_l, W_r)
    acc, accd = _edge_pass(xl, xr, srcp, dstp, att.reshape(-1))
    den = accd.reshape(NC, NPAD, 16)[:, :n, :]
    out = _epilogue(xl[:n], xr[:n], acc[0, :n], acc[1, :n], den[0], den[1],
                    att.reshape(1, -1), bias.reshape(1, -1))
    return out


# speedup vs baseline: 2.4141x; 2.4073x over previous
import dataclasses

import jax
import jax.numpy as jnp
from jax import lax
from jax.experimental import pallas as pl
from jax.experimental.pallas import tpu as pltpu
from jax.experimental.pallas import tpu_sc as plsc

NEG = 0.2
H = 4
HC = 128
NC, NS = 2, 16
NW = NC * NS
EB = 40
NPAD = 10240
DROWS = NPAD // 8


def _mm_body(f_ref, wl_ref, wr_ref, xl_ref, xr_ref):
    f = f_ref[...]
    xl_ref[...] = jnp.dot(f, wl_ref[...], preferred_element_type=jnp.float32)
    xr_ref[...] = jnp.dot(f, wr_ref[...], preferred_element_type=jnp.float32)


def _project(feats, W_l, W_r):
    n, d = feats.shape
    rb = 1000
    return pl.pallas_call(
        _mm_body,
        grid=(n // rb,),
        in_specs=[
            pl.BlockSpec((rb, d), lambda i: (i, 0)),
            pl.BlockSpec((d, HC), lambda i: (0, 0)),
            pl.BlockSpec((d, HC), lambda i: (0, 0)),
        ],
        out_specs=[
            pl.BlockSpec((rb, HC), lambda i: (i, 0)),
            pl.BlockSpec((rb, HC), lambda i: (i, 0)),
        ],
        out_shape=[
            jax.ShapeDtypeStruct((n, HC), jnp.float32),
            jax.ShapeDtypeStruct((n, HC), jnp.float32),
        ],
    )(feats, W_l, W_r)


def _bcast_lane(v, j):
    idx = jnp.full((16, 1), j, jnp.int32)
    dn = lax.GatherDimensionNumbers(
        offset_dims=(), collapsed_slice_dims=(0,), start_index_map=(0,))
    return lax.gather(v, idx, dn, (1,),
                      mode=lax.GatherScatterMode.PROMISE_IN_BOUNDS)


def _edge_pass(xl, xr, src, dst, att_flat):
    e = src.shape[0]
    ept = e // NW
    nblk = ept // EB
    rpt = NPAD // NS

    mesh = plsc.VectorSubcoreMesh(core_axis_name="c", subcore_axis_name="s")

    def body(xl_hbm, xr_hbm, src_hbm, dst_hbm, att_hbm, msg_hbm, den_hbm,
             si0, si1, si2, si3, di0, di1, di2, di3,
             xl0, xl1, xr0, xr1, mb, mb2, didx2, attv,
             acc, accd, sg0, sg1, sx0, sx1):
        c = lax.axis_index("c")
        s = lax.axis_index("s")
        wid = c * NS + s
        ebase = wid * ept

        pltpu.sync_copy(att_hbm, attv)
        att_regs = [attv[pl.ds(16 * k, 16)] for k in range(8)]
        lane = lax.iota(jnp.int32, 16)
        masks = [(lane == h).astype(jnp.float32) for h in range(H)]

        zeros16 = jnp.zeros((16,), jnp.float32)

        @pl.loop(0, EB)
        def _(r):
            for k in range(HC // 16):
                mb[r, pl.ds(16 * k, 16)] = zeros16

        rbase = s * rpt
        for t in range(rpt // EB):
            pltpu.sync_copy(mb.at[pl.ds(0, EB)],
                            acc.at[pl.ds(rbase + t * EB, EB)])
        dbase = s * (DROWS // NS)
        for t in range(DROWS // NS // EB):
            pltpu.sync_copy(mb.at[pl.ds(0, EB)],
                            accd.at[pl.ds(dbase + t * EB, EB)])
        plsc.subcore_barrier()

        si = (si0, si1, si2, si3)
        di = (di0, di1, di2, di3)
        sx = (sx0, sx1)
        sg = (sg0, sg1)
        xlb_ = (xl0, xl1)
        xrb_ = (xr0, xr1)

        def idx_async(j, slot, sem):
            pltpu.async_copy(src_hbm.at[pl.ds(ebase + j * EB, EB)],
                             si[slot], sem)
            pltpu.async_copy(dst_hbm.at[pl.ds(ebase + j * EB, EB)],
                             di[slot], sem)

        def idx_wait(j, slot, sem):
            pltpu.make_async_copy(src_hbm.at[pl.ds(ebase + j * EB, EB)],
                                  si[slot], sem).wait()
            pltpu.make_async_copy(dst_hbm.at[pl.ds(ebase + j * EB, EB)],
                                  di[slot], sem).wait()

        def gathers(slot, p):
            pltpu.async_copy(xl_hbm.at[si[slot]], xlb_[p], sg[p])
            pltpu.async_copy(xr_hbm.at[di[slot]], xrb_[p], sg[p])

        def gather_wait(slot, p):
            pltpu.make_async_copy(xl_hbm.at[si[slot]], xlb_[p],
                                  sg[p]).wait()
            pltpu.make_async_copy(xr_hbm.at[di[slot]], xrb_[p],
                                  sg[p]).wait()

        def process(xlb, xrb, dib):
            @pl.loop(0, EB, step=8)
            def _(g):
                gw = jnp.minimum(g, EB - 16)
                off = g - gw
                dvec = dib[pl.ds(gw, 16)]
                didx2[pl.ds(gw, 16)] = lax.shift_right_logical(dvec, 3)

                @pl.loop(0, 8, unroll=2)
                def _(j):
                    r = g + j
                    den_vec = zeros16
                    for h in range(H):
                        a0 = xlb[r, pl.ds(32 * h, 16)]
                        a1 = xlb[r, pl.ds(32 * h + 16, 16)]
                        b0 = xrb[r, pl.ds(32 * h, 16)]
                        b1 = xrb[r, pl.ds(32 * h + 16, 16)]
                        z0 = a0 + b0
                        z1 = a1 + b1
                        l0 = jnp.maximum(z0, NEG * z0)
                        l1 = jnp.maximum(z1, NEG * z1)
                        t = l0 * att_regs[2 * h] + l1 * att_regs[2 * h + 1]
                        sc = jnp.sum(t)
                        w = jnp.exp(jnp.full((16,), sc, jnp.float32))
                        mb[r, pl.ds(32 * h, 16)] = a0 * w
                        mb[r, pl.ds(32 * h + 16, 16)] = a1 * w
                        den_vec = den_vec + masks[h] * w
                    grp = jnp.bitwise_and(_bcast_lane(dvec, off + j), 7)
                    for k in range(8):
                        vk = jnp.where(grp == k, den_vec, zeros16)
                        mb2[r, pl.ds(16 * k, 16)] = vk

            pltpu.sync_copy(mb, acc.at[dib], add=True)
            pltpu.sync_copy(mb2, accd.at[didx2], add=True)

        pltpu.sync_copy(src_hbm.at[pl.ds(ebase, EB)], si0)
        pltpu.sync_copy(dst_hbm.at[pl.ds(ebase, EB)], di0)
        pltpu.sync_copy(src_hbm.at[pl.ds(ebase + EB, EB)], si1)
        pltpu.sync_copy(dst_hbm.at[pl.ds(ebase + EB, EB)], di1)
        gathers(0, 0)

        @pl.loop(0, nblk - 2, step=4)
        def _(jj):
            for b in range(4):
                j = jj + b
                p = b % 2
                idx_async(j + 2, (b + 2) % 4, sx[p])
                if b == 0:
                    @pl.when(jj >= 1)
                    def _(j=j, b=b, p=p):
                        idx_wait(j + 1, (b + 1) % 4, sx[1 - p])
                else:
                    idx_wait(j + 1, (b + 1) % 4, sx[1 - p])
                gathers((b + 1) % 4, 1 - p)
                gather_wait(b, p)
                process(xlb_[p], xrb_[p], di[b])

        idx_wait(nblk - 1, 1, sx[1])
        gathers(1, 1)
        gather_wait(0, 0)
        process(xl0, xr0, di0)
        gather_wait(1, 1)
        process(xl1, xr1, di1)

        plsc.subcore_barrier()
        pltpu.sync_copy(acc.at[pl.ds(rbase, rpt)],
                        msg_hbm.at[c, pl.ds(rbase, rpt)])
        drpt = DROWS // NS
        pltpu.sync_copy(accd.at[pl.ds(dbase, drpt)],
                        den_hbm.at[c, pl.ds(dbase, drpt)])

    f32 = jnp.float32
    i32 = jnp.int32
    cp = pltpu.CompilerParams()
    if "needs_layout_passes" in pltpu.CompilerParams.__dataclass_fields__:
        cp = dataclasses.replace(cp, needs_layout_passes=False)
    return pl.kernel(
        body,
        compiler_params=cp,
        out_type=(
            jax.ShapeDtypeStruct((NC, NPAD, HC), f32),
            jax.ShapeDtypeStruct((NC, DROWS, HC), f32),
        ),
        mesh=mesh,
        scratch_types=[
            pltpu.VMEM((EB,), i32), pltpu.VMEM((EB,), i32),
            pltpu.VMEM((EB,), i32), pltpu.VMEM((EB,), i32),
            pltpu.VMEM((EB,), i32), pltpu.VMEM((EB,), i32),
            pltpu.VMEM((EB,), i32), pltpu.VMEM((EB,), i32),
            pltpu.VMEM((EB, HC), f32), pltpu.VMEM((EB, HC), f32),
            pltpu.VMEM((EB, HC), f32), pltpu.VMEM((EB, HC), f32),
            pltpu.VMEM((EB, HC), f32), pltpu.VMEM((EB, HC), f32),
            pltpu.VMEM((EB,), i32),
            pltpu.VMEM((HC,), f32),
            pltpu.VMEM_SHARED((NPAD, HC), f32),
            pltpu.VMEM_SHARED((DROWS, HC), f32),
            pltpu.SemaphoreType.DMA, pltpu.SemaphoreType.DMA,
            pltpu.SemaphoreType.DMA, pltpu.SemaphoreType.DMA,
        ],
    )(xl, xr, src, dst, att_flat)


def _post_body(xl_ref, xr_ref, a0_ref, a1_ref, d0_ref, d1_ref, att_ref,
               bias_ref, o_ref):
    xl = xl_ref[...]
    xr = xr_ref[...]
    z = xl + xr
    lk = jnp.maximum(z, NEG * z)
    t = lk * att_ref[...]
    a0 = a0_ref[...]
    a1 = a1_ref[...]
    for h in range(H):
        sl = slice(32 * h, 32 * h + 32)
        s_h = jnp.sum(t[:, sl], axis=1, keepdims=True)
        w_h = jnp.exp(s_h)
        num = a0[:, sl] + a1[:, sl] + w_h * xl[:, sl]
        den = d0_ref[:, h:h + 1] + d1_ref[:, h:h + 1] + w_h
        o_ref[:, sl] = num / den + bias_ref[:, sl]


def _epilogue(xl, xr, a0, a1, d0, d1, att_row, bias_row):
    n = xl.shape[0]
    rb = 1000
    return pl.pallas_call(
        _post_body,
        grid=(n // rb,),
        in_specs=[
            pl.BlockSpec((rb, HC), lambda i: (i, 0)),
            pl.BlockSpec((rb, HC), lambda i: (i, 0)),
            pl.BlockSpec((rb, HC), lambda i: (i, 0)),
            pl.BlockSpec((rb, HC), lambda i: (i, 0)),
            pl.BlockSpec((rb, 16), lambda i: (i, 0)),
            pl.BlockSpec((rb, 16), lambda i: (i, 0)),
            pl.BlockSpec((1, HC), lambda i: (0, 0)),
            pl.BlockSpec((1, HC), lambda i: (0, 0)),
        ],
        out_specs=pl.BlockSpec((rb, HC), lambda i: (i, 0)),
        out_shape=jax.ShapeDtypeStruct((n, HC), jnp.float32),
    )(xl, xr, a0, a1, d0, d1, att_row, bias_row)


def kernel(feats, edges, batches, W_l, W_r, att, bias):
    n = feats.shape[0]
    xl, xr = _project(feats, W_l, W_r)
    acc, accd = _edge_pass(xl, xr, edges[0], edges[1], att.reshape(-1))
    den = accd.reshape(NC, NPAD, 16)[:, :n, :]
    out = _epilogue(xl, xr, acc[0, :n], acc[1, :n], den[0], den[1],
                    att.reshape(1, -1), bias.reshape(1, -1))
    return out
